# baseline (speedup 1.0000x reference)
import functools

import jax
import jax.numpy as jnp
from jax import lax
from jax.experimental import pallas as pl
from jax.experimental.pallas import tpu as pltpu
from jax.experimental.pallas import tpu_sc as plsc

BATCH = 16384
NF = 100
ED = 32
OD = 2 * ED

_info = plsc.get_sparse_core_info()
NC, NS, L = _info.num_cores, _info.num_subcores, _info.num_lanes
NW = NC * NS
BW = BATCH // NW
NB16 = BW // L


def _sc_body(x_hbm, m_hbm, wsp_hbm, bsp_hbm, bps_hbm, dds_hbm, out_hbm,
             x_v, m_v, wsp_v, bsp_v, bps_v, dds_v, ob0, ob1, sem0, sem1):
    wid = lax.axis_index("s") * NC + lax.axis_index("c")
    base = wid * BW

    pltpu.sync_copy(x_hbm.at[:, pl.ds(base, BW)], x_v)
    pltpu.sync_copy(m_hbm.at[:, pl.ds(base, BW)], m_v)
    pltpu.sync_copy(wsp_hbm, wsp_v)
    pltpu.sync_copy(bsp_hbm, bsp_v)
    pltpu.sync_copy(bps_hbm, bps_v)
    pltpu.sync_copy(dds_hbm, dds_v)

    obs = (ob0, ob1)
    sems = (sem0, sem1)
    lane_ids = [jnp.full((L,), i, jnp.int32) for i in range(8)]

    def bcast_lane(v16, idx):
        return lax.gather(
            v16, idx[:, None],
            dimension_numbers=lax.GatherDimensionNumbers(
                offset_dims=(), collapsed_slice_dims=(0,),
                start_index_map=(0,)),
            slice_sizes=(1,),
            mode=lax.GatherScatterMode.PROMISE_IN_BOUNDS)

    def jbody(j, _):
        for dg in range(8):
            p = dg % 2
            value_half = dg < 4
            if value_half:
                sp_a = [wsp_v[pl.ds((dg * 8 + i) * L, L)] for i in range(8)]
                sp_b = [bsp_v[pl.ds((dg * 8 + i) * L, L)] for i in range(8)]
            else:
                t16a = bps_v[pl.ds(j * ED + (dg - 4) * 8, L)]
                t16b = dds_v[pl.ds(j * ED + (dg - 4) * 8, L)]
                sp_a = [bcast_lane(t16b, lane_ids[i]) for i in range(8)]
                sp_b = [bcast_lane(t16a, lane_ids[i]) for i in range(8)]

            if dg >= 2:
                pltpu.make_async_copy(
                    obs[p],
                    out_hbm.at[0, pl.ds(0, 8), pl.ds(base, BW)],
                    sems[p]).wait()
            else:
                @pl.when(j > 0)
                def _w():
                    pltpu.make_async_copy(
                        obs[p],
                        out_hbm.at[0, pl.ds(0, 8), pl.ds(base, BW)],
                        sems[p]).wait()

            ob = obs[p]

            def bbody(b16, _, ob=ob, sp_a=sp_a, sp_b=sp_b,
                      value_half=value_half):
                x16 = x_v[j, pl.ds(b16 * L, L)]
                m16 = m_v[j, pl.ds(b16 * L, L)]
                if value_half:
                    a = x16 - x16 * m16
                    t = 1.0 - m16
                    for i in range(8):
                        ob[i, pl.ds(b16 * L, L)] = a * sp_a[i] + t * sp_b[i]
                else:
                    for i in range(8):
                        ob[i, pl.ds(b16 * L, L)] = m16 * sp_a[i] + sp_b[i]
                return _

            lax.fori_loop(0, NB16, bbody, None)
            pltpu.async_copy(
                ob, out_hbm.at[j, pl.ds(dg * 8, 8), pl.ds(base, BW)],
                sems[p])
        return _

    lax.fori_loop(0, NF, jbody, None)
    for p in range(2):
        pltpu.make_async_copy(
            obs[p], out_hbm.at[0, pl.ds(0, 8), pl.ds(base, BW)],
            sems[p]).wait()


_mesh = plsc.VectorSubcoreMesh(core_axis_name="c", subcore_axis_name="s")

_sc_kernel = functools.partial(
    pl.kernel,
    mesh=_mesh,
    out_type=jax.ShapeDtypeStruct((NF, OD, BATCH), jnp.float32),
    scratch_types=[
        pltpu.VMEM((NF, BW), jnp.float32),
        pltpu.VMEM((NF, BW), jnp.float32),
        pltpu.VMEM((ED * L,), jnp.float32),
        pltpu.VMEM((ED * L,), jnp.float32),
        pltpu.VMEM((NF * ED + L,), jnp.float32),
        pltpu.VMEM((NF * ED + L,), jnp.float32),
        pltpu.VMEM((8, BW), jnp.float32),
        pltpu.VMEM((8, BW), jnp.float32),
        pltpu.SemaphoreType.DMA,
        pltpu.SemaphoreType.DMA,
    ],
)(_sc_body)


def kernel(x_hat, mask, Wv, bv, missing_table, present_table):
    w = Wv[:, 0]
    wsp = jnp.broadcast_to(w[:, None], (ED, L)).reshape(-1)
    bsp = jnp.broadcast_to(bv[:, None], (ED, L)).reshape(-1)
    dds = jnp.pad((missing_table - present_table).reshape(-1), (0, L))
    bps = jnp.pad(present_table.reshape(-1), (0, L))
    out_t = _sc_kernel(x_hat.T, mask.T, wsp, bsp, bps, dds)
    return jnp.transpose(out_t, (2, 0, 1))


# speedup vs baseline: 1.0222x; 1.0222x over previous
import functools

import jax
import jax.numpy as jnp
from jax import lax
from jax.experimental import pallas as pl
from jax.experimental.pallas import tpu as pltpu
from jax.experimental.pallas import tpu_sc as plsc

BATCH = 16384
NF = 100
ED = 32
OD = 2 * ED

_info = plsc.get_sparse_core_info()
NC, NS, L = _info.num_cores, _info.num_subcores, _info.num_lanes
NW = NC * NS
BW = BATCH // NW
NB16 = BW // L
NFP = 104
JBLKS = ((0, 56), (56, 44))
JROWS = 56


def _sc_body(x_hbm, m_hbm, wsp_hbm, bsp_hbm, bps_hbm, dds_hbm, out_hbm,
             x_v, m_v, wsp_v, bsp_v, bps_v, dds_v, ob0, ob1, sem0, sem1):
    wid = lax.axis_index("s") * NC + lax.axis_index("c")
    base = wid * BW

    pltpu.sync_copy(wsp_hbm, wsp_v)
    pltpu.sync_copy(bsp_hbm, bsp_v)
    pltpu.sync_copy(bps_hbm, bps_v)
    pltpu.sync_copy(dds_hbm, dds_v)

    obs = (ob0, ob1)
    sems = (sem0, sem1)
    lane_ids = [jnp.full((L,), i, jnp.int32) for i in range(8)]

    def bcast_lane(v16, idx):
        return lax.gather(
            v16, idx[:, None],
            dimension_numbers=lax.GatherDimensionNumbers(
                offset_dims=(), collapsed_slice_dims=(0,),
                start_index_map=(0,)),
            slice_sizes=(1,),
            mode=lax.GatherScatterMode.PROMISE_IN_BOUNDS)

    for jb, (joff, jcnt) in enumerate(JBLKS):
        rows = min(JROWS, NFP - joff)
        pltpu.sync_copy(
            x_hbm.at[pl.ds(joff, rows), pl.ds(base, BW)],
            x_v.at[pl.ds(0, rows)])
        pltpu.sync_copy(
            m_hbm.at[pl.ds(joff, rows), pl.ds(base, BW)],
            m_v.at[pl.ds(0, rows)])

        def do_j(j2, p, jb, joff=joff):
            j = joff + j2
            ob = obs[p]

            def wait_prev():
                pltpu.make_async_copy(
                    ob, out_hbm.at[0, :, pl.ds(base, BW)], sems[p]).wait()

            if jb == 0:
                @pl.when(j2 >= 2)
                def _w():
                    wait_prev()
            else:
                wait_prev()

            for dg in range(8):
                if dg < 4:
                    sp_a = [wsp_v[pl.ds((dg * 8 + i) * L, L)]
                            for i in range(8)]
                    sp_b = [bsp_v[pl.ds((dg * 8 + i) * L, L)]
                            for i in range(8)]
                else:
                    t16a = bps_v[pl.ds(j * ED + (dg - 4) * 8, L)]
                    t16b = dds_v[pl.ds(j * ED + (dg - 4) * 8, L)]
                    sp_a = [bcast_lane(t16b, lane_ids[i]) for i in range(8)]
                    sp_b = [bcast_lane(t16a, lane_ids[i]) for i in range(8)]

                def bbody(b16, _, ob=ob, sp_a=sp_a, sp_b=sp_b, dg=dg, j2=j2):
                    x16 = x_v[j2, pl.ds(b16 * L, L)]
                    m16 = m_v[j2, pl.ds(b16 * L, L)]
                    if dg < 4:
                        t = 1.0 - m16
                        a = x16 * t
                        for i in range(8):
                            ob[dg * 8 + i, pl.ds(b16 * L, L)] = (
                                a * sp_a[i] + t * sp_b[i])
                    else:
                        for i in range(8):
                            ob[dg * 8 + i, pl.ds(b16 * L, L)] = (
                                m16 * sp_a[i] + sp_b[i])
                    return _

                lax.fori_loop(0, NB16, bbody, None)

            pltpu.async_copy(
                ob, out_hbm.at[j, :, pl.ds(base, BW)], sems[p])

        def jbody(jpair, _, jb=jb):
            for p in range(2):
                do_j(jpair * 2 + p, p, jb)
            return _

        lax.fori_loop(0, jcnt // 2, jbody, None)

    for p in range(2):
        pltpu.make_async_copy(
            obs[p], out_hbm.at[0, :, pl.ds(base, BW)], sems[p]).wait()


_mesh = plsc.VectorSubcoreMesh(core_axis_name="c", subcore_axis_name="s")

_sc_kernel = functools.partial(
    pl.kernel,
    mesh=_mesh,
    out_type=jax.ShapeDtypeStruct((NF, OD, BATCH), jnp.float32),
    scratch_types=[
        pltpu.VMEM((JROWS, BW), jnp.float32),
        pltpu.VMEM((JROWS, BW), jnp.float32),
        pltpu.VMEM((ED * L,), jnp.float32),
        pltpu.VMEM((ED * L,), jnp.float32),
        pltpu.VMEM((NF * ED + L,), jnp.float32),
        pltpu.VMEM((NF * ED + L,), jnp.float32),
        pltpu.VMEM((OD, BW), jnp.float32),
        pltpu.VMEM((OD, BW), jnp.float32),
        pltpu.SemaphoreType.DMA,
        pltpu.SemaphoreType.DMA,
    ],
)(_sc_body)


def kernel(x_hat, mask, Wv, bv, missing_table, present_table):
    w = Wv[:, 0]
    wsp = jnp.broadcast_to(w[:, None], (ED, L)).reshape(-1)
    bsp = jnp.broadcast_to(bv[:, None], (ED, L)).reshape(-1)
    dds = jnp.pad((missing_table - present_table).reshape(-1), (0, L))
    bps = jnp.pad(present_table.reshape(-1), (0, L))
    x_t = jnp.pad(x_hat, ((0, 0), (0, NFP - NF))).T
    m_t = jnp.pad(mask, ((0, 0), (0, NFP - NF))).T
    out_t = _sc_kernel(x_t, m_t, wsp, bsp, bps, dds)
    return jnp.transpose(out_t, (2, 0, 1))


# speedup vs baseline: 1.4207x; 1.3898x over previous
import functools

import jax
import jax.numpy as jnp
from jax import lax
from jax.experimental import pallas as pl
from jax.experimental.pallas import tpu as pltpu
from jax.experimental.pallas import tpu_sc as plsc

BATCH = 16384
NF = 100
ED = 32
OD = 2 * ED

_info = plsc.get_sparse_core_info()
NC, NS, L = _info.num_cores, _info.num_subcores, _info.num_lanes
NW = NC * NS
BW = BATCH // NW
NB16 = BW // L
NFP = 104
JBLKS = ((0, 56), (56, 44))
JROWS = 56


def _sc_body(x_hbm, m_hbm, wsp_hbm, bsp_hbm, bps_hbm, dds_hbm, out_hbm,
             x_v, m_v, wsp_v, bsp_v, bps_v, dds_v, ob0, ob1, sem0, sem1):
    wid = lax.axis_index("s") * NC + lax.axis_index("c")
    base = wid * BW

    pltpu.sync_copy(wsp_hbm, wsp_v)
    pltpu.sync_copy(bsp_hbm, bsp_v)
    pltpu.sync_copy(bps_hbm, bps_v)
    pltpu.sync_copy(dds_hbm, dds_v)

    obs = (ob0, ob1)
    sems = (sem0, sem1)
    lane_ids = [jnp.full((L,), i, jnp.int32) for i in range(8)]

    def bcast_lane(v16, idx):
        return lax.gather(
            v16, idx[:, None],
            dimension_numbers=lax.GatherDimensionNumbers(
                offset_dims=(), collapsed_slice_dims=(0,),
                start_index_map=(0,)),
            slice_sizes=(1,),
            mode=lax.GatherScatterMode.PROMISE_IN_BOUNDS)

    for jb, (joff, jcnt) in enumerate(JBLKS):
        rows = min(JROWS, NFP - joff)
        pltpu.sync_copy(
            x_hbm.at[pl.ds(joff, rows), pl.ds(base, BW)],
            x_v.at[pl.ds(0, rows)])
        pltpu.sync_copy(
            m_hbm.at[pl.ds(joff, rows), pl.ds(base, BW)],
            m_v.at[pl.ds(0, rows)])

        def do_j(j2, p, jb, joff=joff):
            j = joff + j2
            ob = obs[p]

            def wait_prev():
                pltpu.make_async_copy(
                    ob, out_hbm.at[0, :, pl.ds(base, BW)], sems[p]).wait()

            if jb == 0:
                @pl.when(j2 >= 2)
                def _w():
                    wait_prev()
            else:
                wait_prev()

            for dg in range(8):
                if dg < 4:
                    sp_a = [wsp_v[pl.ds((dg * 8 + i) * L, L)]
                            for i in range(8)]
                    sp_b = [bsp_v[pl.ds((dg * 8 + i) * L, L)]
                            for i in range(8)]
                else:
                    t16a = bps_v[pl.ds(j * ED + (dg - 4) * 8, L)]
                    t16b = dds_v[pl.ds(j * ED + (dg - 4) * 8, L)]
                    sp_a = [bcast_lane(t16b, lane_ids[i]) for i in range(8)]
                    sp_b = [bcast_lane(t16a, lane_ids[i]) for i in range(8)]

                @plsc.parallel_loop(0, BW, L, unroll=4)
                def bbody(boff, ob=ob, sp_a=sp_a, sp_b=sp_b, dg=dg, j2=j2):
                    x16 = x_v[j2, pl.ds(boff, L)]
                    m16 = m_v[j2, pl.ds(boff, L)]
                    if dg < 4:
                        t = 1.0 - m16
                        a = x16 * t
                        for i in range(8):
                            ob[dg * 8 + i, pl.ds(boff, L)] = (
                                a * sp_a[i] + t * sp_b[i])
                    else:
                        for i in range(8):
                            ob[dg * 8 + i, pl.ds(boff, L)] = (
                                m16 * sp_a[i] + sp_b[i])

            pltpu.async_copy(
                ob, out_hbm.at[j, :, pl.ds(base, BW)], sems[p])

        def jbody(jpair, _, jb=jb):
            for p in range(2):
                do_j(jpair * 2 + p, p, jb)
            return _

        lax.fori_loop(0, jcnt // 2, jbody, None)

    for p in range(2):
        pltpu.make_async_copy(
            obs[p], out_hbm.at[0, :, pl.ds(base, BW)], sems[p]).wait()


_mesh = plsc.VectorSubcoreMesh(core_axis_name="c", subcore_axis_name="s")

_sc_kernel = functools.partial(
    pl.kernel,
    mesh=_mesh,
    out_type=jax.ShapeDtypeStruct((NF, OD, BATCH), jnp.float32),
    scratch_types=[
        pltpu.VMEM((JROWS, BW), jnp.float32),
        pltpu.VMEM((JROWS, BW), jnp.float32),
        pltpu.VMEM((ED * L,), jnp.float32),
        pltpu.VMEM((ED * L,), jnp.float32),
        pltpu.VMEM((NF * ED + L,), jnp.float32),
        pltpu.VMEM((NF * ED + L,), jnp.float32),
        pltpu.VMEM((OD, BW), jnp.float32),
        pltpu.VMEM((OD, BW), jnp.float32),
        pltpu.SemaphoreType.DMA,
        pltpu.SemaphoreType.DMA,
    ],
)(_sc_body)


def kernel(x_hat, mask, Wv, bv, missing_table, present_table):
    w = Wv[:, 0]
    wsp = jnp.broadcast_to(w[:, None], (ED, L)).reshape(-1)
    bsp = jnp.broadcast_to(bv[:, None], (ED, L)).reshape(-1)
    dds = jnp.pad((missing_table - present_table).reshape(-1), (0, L))
    bps = jnp.pad(present_table.reshape(-1), (0, L))
    x_t = jnp.pad(x_hat, ((0, 0), (0, NFP - NF))).T
    m_t = jnp.pad(mask, ((0, 0), (0, NFP - NF))).T
    out_t = _sc_kernel(x_t, m_t, wsp, bsp, bps, dds)
    return jnp.transpose(out_t, (2, 0, 1))


# speedup vs baseline: 1.4567x; 1.0253x over previous
import functools

import jax
import jax.numpy as jnp
from jax import lax
from jax.experimental import pallas as pl
from jax.experimental.pallas import tpu as pltpu
from jax.experimental.pallas import tpu_sc as plsc

BATCH = 16384
NF = 100
ED = 32
OD = 2 * ED

_info = plsc.get_sparse_core_info()
NC, NS, L = _info.num_cores, _info.num_subcores, _info.num_lanes
NW = NC * NS
BW = BATCH // NW
NB16 = BW // L
NFP = 104
JBLKS = ((0, 56), (56, 44))
JROWS = 56


def _sc_body(x_hbm, m_hbm, wsp_hbm, bsp_hbm, bps_hbm, dds_hbm, out_hbm,
             x_v, m_v, wsp_v, bsp_v, bps_v, dds_v, ob0, ob1, sem0, sem1):
    wid = lax.axis_index("s") * NC + lax.axis_index("c")
    base = wid * BW

    pltpu.sync_copy(wsp_hbm, wsp_v)
    pltpu.sync_copy(bsp_hbm, bsp_v)
    pltpu.sync_copy(bps_hbm, bps_v)
    pltpu.sync_copy(dds_hbm, dds_v)

    obs = (ob0, ob1)
    sems = (sem0, sem1)
    lane_ids = [jnp.full((L,), i, jnp.int32) for i in range(8)]

    def bcast_lane(v16, idx):
        return lax.gather(
            v16, idx[:, None],
            dimension_numbers=lax.GatherDimensionNumbers(
                offset_dims=(), collapsed_slice_dims=(0,),
                start_index_map=(0,)),
            slice_sizes=(1,),
            mode=lax.GatherScatterMode.PROMISE_IN_BOUNDS)

    for jb, (joff, jcnt) in enumerate(JBLKS):
        rows = min(JROWS, NFP - joff)
        pltpu.sync_copy(
            x_hbm.at[pl.ds(joff, rows), pl.ds(base, BW)],
            x_v.at[pl.ds(0, rows)])
        pltpu.sync_copy(
            m_hbm.at[pl.ds(joff, rows), pl.ds(base, BW)],
            m_v.at[pl.ds(0, rows)])

        def do_j(j2, p, jb, joff=joff):
            j = joff + j2
            ob = obs[p]

            def wait_prev():
                pltpu.make_async_copy(
                    ob, out_hbm.at[0, :, pl.ds(base, BW)], sems[p]).wait()

            if jb == 0:
                @pl.when(j2 >= 2)
                def _w():
                    wait_prev()
            else:
                wait_prev()

            for dg in range(8):
                if dg < 4:
                    sp_a = [wsp_v[pl.ds((dg * 8 + i) * L, L)]
                            for i in range(8)]
                    sp_b = [bsp_v[pl.ds((dg * 8 + i) * L, L)]
                            for i in range(8)]
                else:
                    t16a = bps_v[pl.ds(j * ED + (dg - 4) * 8, L)]
                    t16b = dds_v[pl.ds(j * ED + (dg - 4) * 8, L)]
                    sp_a = [bcast_lane(t16b, lane_ids[i]) for i in range(8)]
                    sp_b = [bcast_lane(t16a, lane_ids[i]) for i in range(8)]

                @plsc.parallel_loop(0, BW, 2 * L, unroll=2)
                def bbody(boff, ob=ob, sp_a=sp_a, sp_b=sp_b, dg=dg, j2=j2):
                    for h in range(2):
                        bo = boff + h * L
                        x16 = x_v[j2, pl.ds(bo, L)]
                        m16 = m_v[j2, pl.ds(bo, L)]
                        if dg < 4:
                            t = 1.0 - m16
                            a = x16 * t
                            for i in range(8):
                                ob[dg * 8 + i, pl.ds(bo, L)] = (
                                    a * sp_a[i] + t * sp_b[i])
                        else:
                            for i in range(8):
                                ob[dg * 8 + i, pl.ds(bo, L)] = (
                                    m16 * sp_a[i] + sp_b[i])

            pltpu.async_copy(
                ob, out_hbm.at[j, :, pl.ds(base, BW)], sems[p])

        def jbody(jpair, _, jb=jb):
            for p in range(2):
                do_j(jpair * 2 + p, p, jb)
            return _

        lax.fori_loop(0, jcnt // 2, jbody, None)

    for p in range(2):
        pltpu.make_async_copy(
            obs[p], out_hbm.at[0, :, pl.ds(base, BW)], sems[p]).wait()


_mesh = plsc.VectorSubcoreMesh(core_axis_name="c", subcore_axis_name="s")

_sc_kernel = functools.partial(
    pl.kernel,
    mesh=_mesh,
    out_type=jax.ShapeDtypeStruct((NF, OD, BATCH), jnp.float32),
    scratch_types=[
        pltpu.VMEM((JROWS, BW), jnp.float32),
        pltpu.VMEM((JROWS, BW), jnp.float32),
        pltpu.VMEM((ED * L,), jnp.float32),
        pltpu.VMEM((ED * L,), jnp.float32),
        pltpu.VMEM((NF * ED + L,), jnp.float32),
        pltpu.VMEM((NF * ED + L,), jnp.float32),
        pltpu.VMEM((OD, BW), jnp.float32),
        pltpu.VMEM((OD, BW), jnp.float32),
        pltpu.SemaphoreType.DMA,
        pltpu.SemaphoreType.DMA,
    ],
)(_sc_body)


def kernel(x_hat, mask, Wv, bv, missing_table, present_table):
    w = Wv[:, 0]
    wsp = jnp.broadcast_to(w[:, None], (ED, L)).reshape(-1)
    bsp = jnp.broadcast_to(bv[:, None], (ED, L)).reshape(-1)
    dds = jnp.pad((missing_table - present_table).reshape(-1), (0, L))
    bps = jnp.pad(present_table.reshape(-1), (0, L))
    x_t = jnp.pad(x_hat, ((0, 0), (0, NFP - NF))).T
    m_t = jnp.pad(mask, ((0, 0), (0, NFP - NF))).T
    out_t = _sc_kernel(x_t, m_t, wsp, bsp, bps, dds)
    return jnp.transpose(out_t, (2, 0, 1))
